# baseline (speedup 1.0000x reference)
import functools

import jax
import jax.numpy as jnp
from jax import lax
from jax.experimental import pallas as pl
from jax.experimental.pallas import tpu as pltpu
from jax.experimental.pallas import tpu_sc as plsc

DIM = 64
N_HOPS = 2
N_MEMORY = 20
N_REL = 9
RDIM = 8
BATCH = 1024
KGE_W = 0.01
L2_W = 1e-5

NC = 2
NS = 16
NW = NC * NS

NGRP = BATCH * RDIM
GPW = NGRP // NW
SPW = GPW * N_MEMORY
GPB = 16
SPB = GPB * N_MEMORY
NBLK = GPW // GPB
TRW = 80
TPB = SPB // TRW


def _hop_body(ent, idxh_hbm, idxt_hbm, mr_hbm, rel_hbm, v_hbm,
              o_hbm, parts_hbm,
              idxh, idxt, mrv, vv, relv,
              hb0, hb1, tb0, tb1, psv, psk, logits,
              ob0, ob1, partsv,
              hsem0, hsem1, tsem0, tsem1, osem0, osem1):
    cid = lax.axis_index("c")
    sid = lax.axis_index("s")
    wid = sid * NC + cid
    hbufs = [hb0, hb1]
    tbufs = [tb0, tb1]
    obufs = [ob0, ob1]
    hsems = [hsem0, hsem1]
    tsems = [tsem0, tsem1]
    osems = [osem0, osem1]

    pltpu.sync_copy(idxh_hbm.at[pl.ds(wid * SPW, SPW)], idxh)
    pltpu.sync_copy(idxt_hbm.at[pl.ds(wid * SPW, SPW)], idxt)
    pltpu.sync_copy(mr_hbm.at[pl.ds(wid * SPW, SPW)], mrv.at[pl.ds(0, SPW)])
    pltpu.sync_copy(v_hbm.at[pl.ds(wid * (GPW // RDIM), GPW // RDIM)], vv)
    pltpu.sync_copy(rel_hbm, relv)

    it16 = lax.iota(jnp.int32, 16)

    def fire(blk, q):
        for k in range(TPB):
            sl = pl.ds(blk * SPB + k * TRW, TRW)
            dst = pl.ds(k * TRW, TRW)
            pltpu.make_async_copy(
                ent.at[idxh.at[sl]], hbufs[q].at[dst], hsems[q]).start()
            pltpu.make_async_copy(
                ent.at[idxt.at[sl]], tbufs[q].at[dst], tsems[q]).start()

    def wait(q):
        pltpu.make_async_copy(ent.at[pl.ds(0, SPB)], hbufs[q], hsems[q]).wait()
        pltpu.make_async_copy(ent.at[pl.ds(0, SPB)], tbufs[q], tsems[q]).wait()

    def compute(blk, q, b, carry):
        hb, tb = hbufs[q], tbufs[q]
        ob = obufs[q]
        tail = it16 < (N_MEMORY - 16)

        def st_body(st, carry):
            h2, t2, r2, kg = carry
            mrt = mrv[pl.ds(blk * SPB + st * 16, 16)]
            for m in range(16):
                slot = st * 16 + m
                lg = blk * GPB + slot // N_MEMORY
                vrow = lg // RDIM
                mr_s = mrt[m]
                sv = None
                sk = None
                for jj in range(2):
                    h32 = hb[slot, pl.ds(32 * jj, 32)]
                    t32 = tb[slot, pl.ds(32 * jj, 32)]
                    hu = plsc.unpack(h32, format=plsc.PackFormat.INTERLEAVED,
                                     preferred_element_type=jnp.float32)
                    tu = plsc.unpack(t32, format=plsc.PackFormat.INTERLEAVED,
                                     preferred_element_type=jnp.float32)
                    for sub in range(2):
                        j = 2 * jj + sub
                        hj = hu[sub]
                        tj = tu[sub]
                        dsl = pl.ds(16 * j, 16)
                        rj = plsc.load_gather(
                            relv,
                            [jnp.full((16,), mr_s, jnp.int32), it16 + 16 * j])
                        vj = vv[vrow, dsl]
                        hr = hj * rj
                        pv = hr * vj
                        pk = hr * tj
                        sv = pv if sv is None else sv + pv
                        sk = pk if sk is None else sk + pk
                        h2 = h2 + hj * hj
                        t2 = t2 + tj * tj
                        r2 = r2 + rj * rj
                psv[m, :] = sv
                psk[m, :] = sk
            lv = None
            lk = None
            for l in range(16):
                cl = jnp.full((16,), l, jnp.int32)
                cv = plsc.load_gather(psv, [it16, cl])
                ck = plsc.load_gather(psk, [it16, cl])
                lv = cv if lv is None else lv + cv
                lk = ck if lk is None else lk + ck
            logits[pl.ds(st * 16, 16)] = lv
            kg = kg + 1.0 / (1.0 + jnp.exp(-lk))
            return (h2, t2, r2, kg)

        carry = lax.fori_loop(0, SPB // 16, st_body, carry)

        @pl.when(b >= 2)
        def _():
            pltpu.make_async_copy(
                o_hbm.at[pl.ds(0, GPB)], ob, osems[q]).wait()

        def g_body(g, c):
            base = g * N_MEMORY
            l0 = logits[pl.ds(base, 16)]
            l1 = logits[pl.ds(base + 16, 16)]
            m0 = jnp.max(l0)
            m1 = jnp.max(jnp.where(tail, l1, -3e38))
            mx = jnp.maximum(m0, m1)
            e0 = jnp.exp(l0 - mx)
            e1 = jnp.where(tail, jnp.exp(l1 - mx), 0.0)
            s = jnp.full((16,), jnp.sum(e0) + jnp.sum(e1), jnp.float32)
            inv = 1.0 / s
            p0 = e0 * inv
            p1 = e1 * inv
            oj = [None] * 4
            for m in range(N_MEMORY):
                pm = p0[m] if m < 16 else p1[m - 16]
                for jj in range(2):
                    t32 = tb[base + m, pl.ds(32 * jj, 32)]
                    tu = plsc.unpack(t32, format=plsc.PackFormat.INTERLEAVED,
                                     preferred_element_type=jnp.float32)
                    for sub in range(2):
                        j = 2 * jj + sub
                        term = pm * tu[sub]
                        oj[j] = term if oj[j] is None else oj[j] + term
            for j in range(4):
                ob[g, pl.ds(16 * j, 16)] = oj[j]
            return c

        lax.fori_loop(0, GPB, g_body, 0)
        pltpu.make_async_copy(
            ob, o_hbm.at[pl.ds(wid * GPW + b * GPB, GPB)], osems[q]).start()
        return carry

    fire(0, 0)
    zero = jnp.zeros((16,), jnp.float32)

    def pair(i, carry):
        for q in range(2):
            b = 2 * i + q
            if q == 0:
                fire_blk = b + 1
                fire(fire_blk, 1)
            else:
                @pl.when(i < NBLK // 2 - 1)
                def _():
                    fire(b + 1, 0)
            wait(q)
            carry = compute(b, q, b, carry)
        return carry

    h2, t2, r2, kg = lax.fori_loop(0, NBLK // 2, pair, (zero, zero, zero, zero))

    for q in range(2):
        pltpu.make_async_copy(
            o_hbm.at[pl.ds(0, GPB)], obufs[q], osems[q]).wait()

    partsv[pl.ds(0, 16)] = kg
    partsv[pl.ds(16, 16)] = h2
    partsv[pl.ds(32, 16)] = t2
    partsv[pl.ds(48, 16)] = r2
    pltpu.sync_copy(partsv, parts_hbm.at[wid])


@functools.cache
def _hop_kernel():
    return functools.partial(
        pl.kernel,
        out_type=(jax.ShapeDtypeStruct((NGRP, DIM), jnp.float32),
                  jax.ShapeDtypeStruct((NW, 64), jnp.float32)),
        mesh=plsc.VectorSubcoreMesh(core_axis_name="c", subcore_axis_name="s"),
        compiler_params=pltpu.CompilerParams(use_tc_tiling_on_sc=False, needs_layout_passes=False),
        scratch_types=(
            [pltpu.VMEM((SPW,), jnp.int32)] * 2
            + [pltpu.VMEM((SPW + 16,), jnp.int32)]
            + [pltpu.VMEM((GPW // RDIM, DIM), jnp.float32)]
            + [pltpu.VMEM((N_REL, DIM), jnp.float32)]
            + [pltpu.VMEM((SPB, DIM), jnp.bfloat16)] * 4
            + [pltpu.VMEM((16, 16), jnp.float32)] * 2
            + [pltpu.VMEM((SPB + 32,), jnp.float32)]
            + [pltpu.VMEM((GPB, DIM), jnp.float32)] * 2
            + [pltpu.VMEM((64,), jnp.float32)]
            + [pltpu.SemaphoreType.DMA] * 6
        ),
    )(_hop_body)


def _hop_call(ent, idx_h, idx_t, mr, rel, v):
    return _hop_kernel()(ent, idx_h, idx_t, mr, rel, v)


IPW = 2 * BATCH // NW


def _items_body(idx_hbm, ent, out_hbm, idxv, rows, gsem):
    cid = lax.axis_index("c")
    sid = lax.axis_index("s")
    wid = sid * NC + cid
    pltpu.sync_copy(idx_hbm.at[pl.ds(wid * IPW, IPW)], idxv)
    pltpu.make_async_copy(ent.at[idxv], rows, gsem).start()
    pltpu.make_async_copy(ent.at[pl.ds(0, IPW)], rows, gsem).wait()
    pltpu.sync_copy(rows, out_hbm.at[pl.ds(wid * IPW, IPW)])


@functools.cache
def _items_kernel():
    return functools.partial(
        pl.kernel,
        out_type=jax.ShapeDtypeStruct((2 * BATCH, DIM), jnp.float32),
        mesh=plsc.VectorSubcoreMesh(core_axis_name="c", subcore_axis_name="s"),
        compiler_params=pltpu.CompilerParams(use_tc_tiling_on_sc=False, needs_layout_passes=False),
        scratch_types=[
            pltpu.VMEM((IPW,), jnp.int32),
            pltpu.VMEM((IPW, DIM), jnp.float32),
            pltpu.SemaphoreType.DMA,
        ],
    )(_items_body)


def _attn_agg(o2, w1, w2):
    o = o2.reshape(BATCH, RDIM, DIM)
    u = jnp.sum(w1 * w2[None, :], axis=1)
    att = jnp.maximum(jnp.sum(o * u[None, None, :], axis=-1), 0.0)
    att = att - jnp.max(att, axis=-1, keepdims=True)
    e = jnp.exp(att)
    an = e / jnp.sum(e, axis=-1, keepdims=True)
    return jnp.sum(o * an[..., None], axis=1)


def _tc_hop_body(o_ref, v_ref, w1_ref, w2_ref, t_ref, vn_ref, oa_ref):
    oa = _attn_agg(o_ref[...], w1_ref[...], w2_ref[...])
    oa_ref[...] = oa
    vn_ref[...] = jnp.dot(v_ref[...] + oa, t_ref[...],
                          preferred_element_type=jnp.float32)


def _tc_hop(o2, v, w1, w2, tmat):
    return pl.pallas_call(
        _tc_hop_body,
        out_shape=(jax.ShapeDtypeStruct((BATCH, DIM), jnp.float32),
                   jax.ShapeDtypeStruct((BATCH, DIM), jnp.float32)),
    )(o2, v, w1, w2, tmat)


def _tc_final_body(o_ref, w1_ref, w2_ref, items_ref, neg_ref, oa0_ref,
                   p0_ref, p1_ref, t_ref, out_ref):
    oa1 = _attn_agg(o_ref[...], w1_ref[...], w2_ref[...])
    y = oa0_ref[...] + oa1
    ps = jnp.sum(items_ref[...] * y, axis=1)
    ns = jnp.sum(neg_ref[...] * y, axis=1)
    d = ps - ns
    ls = jnp.minimum(d, 0.0) - jnp.log(1.0 + jnp.exp(-jnp.abs(d)))
    mf = -jnp.sum(ls) / BATCH
    p = p0_ref[...] + p1_ref[...]
    kge = jnp.sum(p[:, 0:16]) / (NGRP * N_MEMORY)
    l2 = jnp.sum(p[:, 16:64]) + jnp.sum(t_ref[...] * t_ref[...])
    out_ref[0] = mf - KGE_W * kge + L2_W * l2


def _tc_final(o2, w1, w2, items, neg, oa0, p0, p1, tmat):
    return pl.pallas_call(
        _tc_final_body,
        out_specs=pl.BlockSpec(memory_space=pltpu.SMEM),
        out_shape=jax.ShapeDtypeStruct((1,), jnp.float32),
    )(o2, w1, w2, items, neg, oa0, p0, p1, tmat)


_PERM = (list(range(0, 32, 2)) + list(range(1, 32, 2))
         + list(range(32, 64, 2)) + list(range(33, 64, 2)))


def kernel(pos_items, neg_items, memories_h, memories_r, memories_t,
           entity_emb, relation_emb, transform_matrix, att_w1, att_w2):
    perm = jnp.array(_PERM, dtype=jnp.int32)
    entb = entity_emb.astype(jnp.bfloat16)
    relp = relation_emb[:, perm]
    tpp = transform_matrix[perm][:, perm]
    w1p = att_w1[:, perm, :]

    item_idx = jnp.concatenate([pos_items.astype(jnp.int32),
                                neg_items.astype(jnp.int32)])
    item_rows = _items_kernel()(item_idx, entity_emb)[:, perm]
    items = item_rows[:BATCH]
    negr = item_rows[BATCH:]

    w2 = att_w2.reshape(N_HOPS, DIM)
    ih = [memories_h[h].reshape(-1).astype(jnp.int32) for h in range(N_HOPS)]
    it = [memories_t[h].reshape(-1).astype(jnp.int32) for h in range(N_HOPS)]
    mr = [memories_r[h].reshape(-1).astype(jnp.int32) for h in range(N_HOPS)]

    o0, p0 = _hop_call(entb, ih[0], it[0], mr[0], relp, items)
    v1, oa0 = _tc_hop(o0, items, w1p[0], w2[0], tpp)
    o1, p1 = _hop_call(entb, ih[1], it[1], mr[1], relp, v1)
    out = _tc_final(o1, w1p[1], w2[1], items, negr, oa0, p0, p1, tpp)
    return out[0]


# speedup vs baseline: 1.1868x; 1.1868x over previous
import functools

import jax
import jax.numpy as jnp
from jax import lax
from jax.experimental import pallas as pl
from jax.experimental.pallas import tpu as pltpu
from jax.experimental.pallas import tpu_sc as plsc

DIM = 64
N_HOPS = 2
N_MEMORY = 20
N_REL = 9
RDIM = 8
BATCH = 1024
KGE_W = 0.01
L2_W = 1e-5

NC = 2
NS = 16
NW = NC * NS

NGRP = BATCH * RDIM
GPW = NGRP // NW
SPW = GPW * N_MEMORY
GPB = 16
SPB = GPB * N_MEMORY
NBLK = GPW // GPB
TRW = 80
TPB = SPB // TRW
VPW = GPW // RDIM


def _hop_body(first, *refs):
    if first:
        (ent, idxh_hbm, idxt_hbm, mr_hbm, rel_hbm, iidx_hbm,
         o_hbm, parts_hbm, iout_hbm,
         idxh, idxt, mrv, vv, relv, ibidx, nbuf,
         hb0, hb1, tb0, tb1, psv, psk, logits, ob0, ob1, partsv,
         hsem0, hsem1, tsem0, tsem1, osem0, osem1, isem) = refs
    else:
        (ent, idxh_hbm, idxt_hbm, mr_hbm, rel_hbm, v_hbm,
         o_hbm, parts_hbm,
         idxh, idxt, mrv, vv, relv,
         hb0, hb1, tb0, tb1, psv, psk, logits, ob0, ob1, partsv,
         hsem0, hsem1, tsem0, tsem1, osem0, osem1) = refs
    cid = lax.axis_index("c")
    sid = lax.axis_index("s")
    wid = sid * NC + cid
    hbufs = [hb0, hb1]
    tbufs = [tb0, tb1]
    obufs = [ob0, ob1]
    hsems = [hsem0, hsem1]
    tsems = [tsem0, tsem1]
    osems = [osem0, osem1]

    if first:
        pltpu.sync_copy(iidx_hbm.at[pl.ds(wid * 2 * VPW, 2 * VPW)], ibidx)
        ig0 = pltpu.make_async_copy(ent.at[ibidx.at[pl.ds(0, VPW)]], vv, isem)
        ig1 = pltpu.make_async_copy(
            ent.at[ibidx.at[pl.ds(VPW, VPW)]], nbuf, isem)
        ig0.start()
        ig1.start()
    else:
        pltpu.sync_copy(v_hbm.at[pl.ds(wid * VPW, VPW)], vv)
    pltpu.sync_copy(idxh_hbm.at[pl.ds(wid * SPW, SPW)], idxh)
    pltpu.sync_copy(idxt_hbm.at[pl.ds(wid * SPW, SPW)], idxt)
    pltpu.sync_copy(mr_hbm.at[pl.ds(wid * SPW, SPW)], mrv.at[pl.ds(0, SPW)])
    pltpu.sync_copy(rel_hbm, relv)

    it16 = lax.iota(jnp.int32, 16)

    def fire(blk, q):
        for k in range(TPB):
            sl = pl.ds(blk * SPB + k * TRW, TRW)
            dst = pl.ds(k * TRW, TRW)
            pltpu.make_async_copy(
                ent.at[idxh.at[sl]], hbufs[q].at[dst], hsems[q]).start()
            pltpu.make_async_copy(
                ent.at[idxt.at[sl]], tbufs[q].at[dst], tsems[q]).start()

    def wait(q):
        pltpu.make_async_copy(ent.at[pl.ds(0, SPB)], hbufs[q], hsems[q]).wait()
        pltpu.make_async_copy(ent.at[pl.ds(0, SPB)], tbufs[q], tsems[q]).wait()

    def compute(blk, q, b, carry):
        hb, tb = hbufs[q], tbufs[q]
        ob = obufs[q]
        tail = it16 < (N_MEMORY - 16)

        def st_body(st, carry):
            h2, t2, r2, kg = carry
            mrt = mrv[pl.ds(blk * SPB + st * 16, 16)]
            for m in range(16):
                slot = st * 16 + m
                lg = blk * GPB + slot // N_MEMORY
                vrow = lg // RDIM
                mr_s = mrt[m]
                sv = None
                sk = None
                for j in range(4):
                    dsl = pl.ds(16 * j, 16)
                    hj = hb[slot, dsl]
                    tj = tb[slot, dsl]
                    rj = plsc.load_gather(
                        relv, [jnp.full((16,), mr_s, jnp.int32), it16 + 16 * j])
                    vj = vv[vrow, dsl]
                    hr = hj * rj
                    pv = hr * vj
                    pk = hr * tj
                    sv = pv if sv is None else sv + pv
                    sk = pk if sk is None else sk + pk
                    h2 = h2 + hj * hj
                    t2 = t2 + tj * tj
                    r2 = r2 + rj * rj
                psv[m, :] = sv
                psk[m, :] = sk
            lv = None
            lk = None
            for l in range(16):
                cl = jnp.full((16,), l, jnp.int32)
                cv = plsc.load_gather(psv, [it16, cl])
                ck = plsc.load_gather(psk, [it16, cl])
                lv = cv if lv is None else lv + cv
                lk = ck if lk is None else lk + ck
            logits[pl.ds(st * 16, 16)] = lv
            kg = kg + 1.0 / (1.0 + jnp.exp(-lk))
            return (h2, t2, r2, kg)

        carry = lax.fori_loop(0, SPB // 16, st_body, carry)

        @pl.when(b >= 2)
        def _():
            pltpu.make_async_copy(
                ent.at[pl.ds(0, GPB)], ob, osems[q]).wait()

        def g_body(g, c):
            base = g * N_MEMORY
            l0 = logits[pl.ds(base, 16)]
            l1 = logits[pl.ds(base + 16, 16)]
            m0 = jnp.max(l0)
            m1 = jnp.max(jnp.where(tail, l1, -3e38))
            mx = jnp.maximum(m0, m1)
            e0 = jnp.exp(l0 - mx)
            e1 = jnp.where(tail, jnp.exp(l1 - mx), 0.0)
            s = jnp.full((16,), jnp.sum(e0) + jnp.sum(e1), jnp.float32)
            inv = 1.0 / s
            p0 = e0 * inv
            p1 = e1 * inv
            for j in range(4):
                dsl = pl.ds(16 * j, 16)
                oj = None
                for m in range(N_MEMORY):
                    pm = p0[m] if m < 16 else p1[m - 16]
                    term = pm * tb[base + m, dsl]
                    oj = term if oj is None else oj + term
                ob[g, dsl] = oj
            return c

        lax.fori_loop(0, GPB, g_body, 0)
        pltpu.make_async_copy(
            ob, o_hbm.at[pl.ds(wid * GPW + b * GPB, GPB)], osems[q]).start()
        return carry

    fire(0, 0)
    if first:
        ig0.wait()
        ig1.wait()
        ist0 = pltpu.make_async_copy(
            vv, iout_hbm.at[pl.ds(wid * 2 * VPW, VPW)], isem)
        ist1 = pltpu.make_async_copy(
            nbuf, iout_hbm.at[pl.ds(wid * 2 * VPW + VPW, VPW)], isem)
        ist0.start()
        ist1.start()
    zero = jnp.zeros((16,), jnp.float32)

    def pair(i, carry):
        for q in range(2):
            b = 2 * i + q
            if q == 0:
                fire_blk = b + 1
                fire(fire_blk, 1)
            else:
                @pl.when(i < NBLK // 2 - 1)
                def _():
                    fire(b + 1, 0)
            wait(q)
            carry = compute(b, q, b, carry)
        return carry

    h2, t2, r2, kg = lax.fori_loop(0, NBLK // 2, pair, (zero, zero, zero, zero))

    for q in range(2):
        pltpu.make_async_copy(ent.at[pl.ds(0, GPB)], obufs[q], osems[q]).wait()

    partsv[pl.ds(0, 16)] = kg
    partsv[pl.ds(16, 16)] = h2
    partsv[pl.ds(32, 16)] = t2
    partsv[pl.ds(48, 16)] = r2
    pltpu.sync_copy(partsv, parts_hbm.at[wid])
    if first:
        ist0.wait()
        ist1.wait()


@functools.cache
def _hop_kernel(first):
    out_type = [jax.ShapeDtypeStruct((NGRP, DIM), jnp.float32),
                jax.ShapeDtypeStruct((NW, 64), jnp.float32)]
    if first:
        out_type.append(jax.ShapeDtypeStruct((2 * BATCH, DIM), jnp.float32))
    scratch = (
        [pltpu.VMEM((SPW,), jnp.int32)] * 2
        + [pltpu.VMEM((SPW + 16,), jnp.int32)]
        + [pltpu.VMEM((VPW, DIM), jnp.float32)]
        + [pltpu.VMEM((N_REL, DIM), jnp.float32)]
        + ([pltpu.VMEM((2 * VPW,), jnp.int32),
            pltpu.VMEM((VPW, DIM), jnp.float32)]
           if first else [])
        + [pltpu.VMEM((SPB, DIM), jnp.float32)] * 4
        + [pltpu.VMEM((16, 16), jnp.float32)] * 2
        + [pltpu.VMEM((SPB + 32,), jnp.float32)]
        + [pltpu.VMEM((GPB, DIM), jnp.float32)] * 2
        + [pltpu.VMEM((64,), jnp.float32)]
        + [pltpu.SemaphoreType.DMA] * (7 if first else 6)
    )
    return functools.partial(
        pl.kernel,
        out_type=tuple(out_type),
        mesh=plsc.VectorSubcoreMesh(core_axis_name="c", subcore_axis_name="s"),
        compiler_params=pltpu.CompilerParams(
            use_tc_tiling_on_sc=False, needs_layout_passes=False),
        scratch_types=scratch,
    )(functools.partial(_hop_body, first))


def _attn_agg(o2, w1, w2):
    o = o2.reshape(BATCH, RDIM, DIM)
    u = jnp.sum(w1 * w2[None, :], axis=1)
    att = jnp.maximum(jnp.sum(o * u[None, None, :], axis=-1), 0.0)
    att = att - jnp.max(att, axis=-1, keepdims=True)
    e = jnp.exp(att)
    an = e / jnp.sum(e, axis=-1, keepdims=True)
    return jnp.sum(o * an[..., None], axis=1)


def _tc_hop_body(o_ref, v_ref, w1_ref, w2_ref, t_ref, vn_ref, oa_ref):
    oa = _attn_agg(o_ref[...], w1_ref[...], w2_ref[...])
    oa_ref[...] = oa
    vn_ref[...] = jnp.dot(v_ref[...] + oa, t_ref[...],
                          preferred_element_type=jnp.float32)


def _tc_hop(o2, v, w1, w2, tmat):
    return pl.pallas_call(
        _tc_hop_body,
        out_shape=(jax.ShapeDtypeStruct((BATCH, DIM), jnp.float32),
                   jax.ShapeDtypeStruct((BATCH, DIM), jnp.float32)),
    )(o2, v, w1, w2, tmat)


def _tc_final_body(o_ref, w1_ref, w2_ref, items_ref, neg_ref, oa0_ref,
                   p0_ref, p1_ref, t_ref, out_ref):
    oa1 = _attn_agg(o_ref[...], w1_ref[...], w2_ref[...])
    y = oa0_ref[...] + oa1
    ps = jnp.sum(items_ref[...] * y, axis=1)
    ns = jnp.sum(neg_ref[...] * y, axis=1)
    d = ps - ns
    ls = jnp.minimum(d, 0.0) - jnp.log(1.0 + jnp.exp(-jnp.abs(d)))
    mf = -jnp.sum(ls) / BATCH
    p = p0_ref[...] + p1_ref[...]
    kge = jnp.sum(p[:, 0:16]) / (NGRP * N_MEMORY)
    l2 = jnp.sum(p[:, 16:64]) + jnp.sum(t_ref[...] * t_ref[...])
    out_ref[0] = mf - KGE_W * kge + L2_W * l2


def _tc_final(o2, w1, w2, items, neg, oa0, p0, p1, tmat):
    return pl.pallas_call(
        _tc_final_body,
        out_specs=pl.BlockSpec(memory_space=pltpu.SMEM),
        out_shape=jax.ShapeDtypeStruct((1,), jnp.float32),
    )(o2, w1, w2, items, neg, oa0, p0, p1, tmat)


def kernel(pos_items, neg_items, memories_h, memories_r, memories_t,
           entity_emb, relation_emb, transform_matrix, att_w1, att_w2):
    ent = entity_emb
    rel = relation_emb
    item_idx = jnp.concatenate(
        [pos_items.astype(jnp.int32).reshape(NW, VPW),
         neg_items.astype(jnp.int32).reshape(NW, VPW)], axis=1).reshape(-1)
    w2 = att_w2.reshape(N_HOPS, DIM)

    ih0 = memories_h[0].reshape(-1).astype(jnp.int32)
    it0 = memories_t[0].reshape(-1).astype(jnp.int32)
    mr0 = memories_r[0].reshape(-1).astype(jnp.int32)
    o0, p0, irows = _hop_kernel(True)(ent, ih0, it0, mr0, rel, item_idx)
    ir = irows.reshape(NW, 2, VPW, DIM)
    items = ir[:, 0].reshape(BATCH, DIM)
    negr = ir[:, 1].reshape(BATCH, DIM)

    v1, oa0 = _tc_hop(o0, items, att_w1[0], w2[0], transform_matrix)
    ih1 = memories_h[1].reshape(-1).astype(jnp.int32)
    it1 = memories_t[1].reshape(-1).astype(jnp.int32)
    mr1 = memories_r[1].reshape(-1).astype(jnp.int32)
    o1, p1 = _hop_kernel(False)(ent, ih1, it1, mr1, rel, v1)
    out = _tc_final(o1, att_w1[1], w2[1], items, negr, oa0, p0, p1,
                    transform_matrix)
    return out[0]


# speedup vs baseline: 1.1906x; 1.0032x over previous
import functools

import jax
import jax.numpy as jnp
from jax import lax
from jax.experimental import pallas as pl
from jax.experimental.pallas import tpu as pltpu
from jax.experimental.pallas import tpu_sc as plsc

DIM = 64
N_HOPS = 2
N_MEMORY = 20
N_REL = 9
RDIM = 8
BATCH = 1024
KGE_W = 0.01
L2_W = 1e-5

NC = 2
NS = 16
NW = NC * NS

NGRP = BATCH * RDIM
GPW = NGRP // NW
SPW = GPW * N_MEMORY
GPB = 16
SPB = GPB * N_MEMORY
NBLK = GPW // GPB
TRW = 160
TPB = SPB // TRW
VPW = GPW // RDIM


def _hop_body(first, *refs):
    if first:
        (ent, idxh_hbm, idxt_hbm, mr_hbm, rel_hbm, iidx_hbm,
         o_hbm, parts_hbm, iout_hbm,
         idxh, idxt, mrv, vv, relv, ibidx, nbuf,
         hb0, hb1, tb0, tb1, psv, psk, logits, ob0, ob1, partsv,
         hsem0, hsem1, tsem0, tsem1, osem0, osem1, isem) = refs
    else:
        (ent, idxh_hbm, idxt_hbm, mr_hbm, rel_hbm, v_hbm,
         o_hbm, parts_hbm,
         idxh, idxt, mrv, vv, relv,
         hb0, hb1, tb0, tb1, psv, psk, logits, ob0, ob1, partsv,
         hsem0, hsem1, tsem0, tsem1, osem0, osem1) = refs
    cid = lax.axis_index("c")
    sid = lax.axis_index("s")
    wid = sid * NC + cid
    hbufs = [hb0, hb1]
    tbufs = [tb0, tb1]
    obufs = [ob0, ob1]
    hsems = [hsem0, hsem1]
    tsems = [tsem0, tsem1]
    osems = [osem0, osem1]

    if first:
        pltpu.sync_copy(iidx_hbm.at[pl.ds(wid * 2 * VPW, 2 * VPW)], ibidx)
        ig0 = pltpu.make_async_copy(ent.at[ibidx.at[pl.ds(0, VPW)]], vv, isem)
        ig1 = pltpu.make_async_copy(
            ent.at[ibidx.at[pl.ds(VPW, VPW)]], nbuf, isem)
        ig0.start()
        ig1.start()
    else:
        pltpu.sync_copy(v_hbm.at[pl.ds(wid * VPW, VPW)], vv)
    pltpu.sync_copy(idxh_hbm.at[pl.ds(wid * SPW, SPW)], idxh)
    pltpu.sync_copy(idxt_hbm.at[pl.ds(wid * SPW, SPW)], idxt)
    pltpu.sync_copy(mr_hbm.at[pl.ds(wid * SPW, SPW)], mrv.at[pl.ds(0, SPW)])
    pltpu.sync_copy(rel_hbm, relv)

    it16 = lax.iota(jnp.int32, 16)

    def fire(blk, q):
        for k in range(TPB):
            sl = pl.ds(blk * SPB + k * TRW, TRW)
            dst = pl.ds(k * TRW, TRW)
            pltpu.make_async_copy(
                ent.at[idxh.at[sl]], hbufs[q].at[dst], hsems[q]).start()
            pltpu.make_async_copy(
                ent.at[idxt.at[sl]], tbufs[q].at[dst], tsems[q]).start()

    def wait(q):
        pltpu.make_async_copy(ent.at[pl.ds(0, SPB)], hbufs[q], hsems[q]).wait()
        pltpu.make_async_copy(ent.at[pl.ds(0, SPB)], tbufs[q], tsems[q]).wait()

    def compute(blk, q, b, carry):
        hb, tb = hbufs[q], tbufs[q]
        ob = obufs[q]
        tail = it16 < (N_MEMORY - 16)

        def st_body(st, carry):
            h2, t2, r2, kg = carry
            mrt = mrv[pl.ds(blk * SPB + st * 16, 16)]
            for m in range(16):
                slot = st * 16 + m
                lg = blk * GPB + slot // N_MEMORY
                vrow = lg // RDIM
                mr_s = mrt[m]
                sv = None
                sk = None
                for j in range(4):
                    dsl = pl.ds(16 * j, 16)
                    hj = hb[slot, dsl]
                    tj = tb[slot, dsl]
                    rj = plsc.load_gather(
                        relv, [jnp.full((16,), mr_s, jnp.int32), it16 + 16 * j])
                    vj = vv[vrow, dsl]
                    hr = hj * rj
                    pv = hr * vj
                    pk = hr * tj
                    sv = pv if sv is None else sv + pv
                    sk = pk if sk is None else sk + pk
                    h2 = h2 + hj * hj
                    t2 = t2 + tj * tj
                    r2 = r2 + rj * rj
                psv[m, :] = sv
                psk[m, :] = sk
            lv = None
            lk = None
            for l in range(16):
                cl = jnp.full((16,), l, jnp.int32)
                cv = plsc.load_gather(psv, [it16, cl])
                ck = plsc.load_gather(psk, [it16, cl])
                lv = cv if lv is None else lv + cv
                lk = ck if lk is None else lk + ck
            logits[pl.ds(st * 16, 16)] = lv
            kg = kg + 1.0 / (1.0 + jnp.exp(-lk))
            return (h2, t2, r2, kg)

        carry = lax.fori_loop(0, SPB // 16, st_body, carry)

        @pl.when(b >= 2)
        def _():
            pltpu.make_async_copy(
                ent.at[pl.ds(0, GPB)], ob, osems[q]).wait()

        def g_body(g, c):
            base = g * N_MEMORY
            l0 = logits[pl.ds(base, 16)]
            l1 = logits[pl.ds(base + 16, 16)]
            m0 = jnp.max(l0)
            m1 = jnp.max(jnp.where(tail, l1, -3e38))
            mx = jnp.maximum(m0, m1)
            e0 = jnp.exp(l0 - mx)
            e1 = jnp.where(tail, jnp.exp(l1 - mx), 0.0)
            s = jnp.full((16,), jnp.sum(e0) + jnp.sum(e1), jnp.float32)
            inv = 1.0 / s
            p0 = e0 * inv
            p1 = e1 * inv
            for j in range(4):
                dsl = pl.ds(16 * j, 16)
                oj = None
                for m in range(N_MEMORY):
                    pm = p0[m] if m < 16 else p1[m - 16]
                    term = pm * tb[base + m, dsl]
                    oj = term if oj is None else oj + term
                ob[g, dsl] = oj
            return c

        lax.fori_loop(0, GPB, g_body, 0)
        pltpu.make_async_copy(
            ob, o_hbm.at[pl.ds(wid * GPW + b * GPB, GPB)], osems[q]).start()
        return carry

    fire(0, 0)
    if first:
        ig0.wait()
        ig1.wait()
        ist0 = pltpu.make_async_copy(
            vv, iout_hbm.at[pl.ds(wid * 2 * VPW, VPW)], isem)
        ist1 = pltpu.make_async_copy(
            nbuf, iout_hbm.at[pl.ds(wid * 2 * VPW + VPW, VPW)], isem)
        ist0.start()
        ist1.start()
    zero = jnp.zeros((16,), jnp.float32)

    def pair(i, carry):
        for q in range(2):
            b = 2 * i + q
            if q == 0:
                fire_blk = b + 1
                fire(fire_blk, 1)
            else:
                @pl.when(i < NBLK // 2 - 1)
                def _():
                    fire(b + 1, 0)
            wait(q)
            carry = compute(b, q, b, carry)
        return carry

    h2, t2, r2, kg = lax.fori_loop(0, NBLK // 2, pair, (zero, zero, zero, zero))

    for q in range(2):
        pltpu.make_async_copy(ent.at[pl.ds(0, GPB)], obufs[q], osems[q]).wait()

    partsv[pl.ds(0, 16)] = kg
    partsv[pl.ds(16, 16)] = h2
    partsv[pl.ds(32, 16)] = t2
    partsv[pl.ds(48, 16)] = r2
    pltpu.sync_copy(partsv, parts_hbm.at[wid])
    if first:
        ist0.wait()
        ist1.wait()


@functools.cache
def _hop_kernel(first):
    out_type = [jax.ShapeDtypeStruct((NGRP, DIM), jnp.float32),
                jax.ShapeDtypeStruct((NW, 64), jnp.float32)]
    if first:
        out_type.append(jax.ShapeDtypeStruct((2 * BATCH, DIM), jnp.float32))
    scratch = (
        [pltpu.VMEM((SPW,), jnp.int32)] * 2
        + [pltpu.VMEM((SPW + 16,), jnp.int32)]
        + [pltpu.VMEM((VPW, DIM), jnp.float32)]
        + [pltpu.VMEM((N_REL, DIM), jnp.float32)]
        + ([pltpu.VMEM((2 * VPW,), jnp.int32),
            pltpu.VMEM((VPW, DIM), jnp.float32)]
           if first else [])
        + [pltpu.VMEM((SPB, DIM), jnp.float32)] * 4
        + [pltpu.VMEM((16, 16), jnp.float32)] * 2
        + [pltpu.VMEM((SPB + 32,), jnp.float32)]
        + [pltpu.VMEM((GPB, DIM), jnp.float32)] * 2
        + [pltpu.VMEM((64,), jnp.float32)]
        + [pltpu.SemaphoreType.DMA] * (7 if first else 6)
    )
    return functools.partial(
        pl.kernel,
        out_type=tuple(out_type),
        mesh=plsc.VectorSubcoreMesh(core_axis_name="c", subcore_axis_name="s"),
        compiler_params=pltpu.CompilerParams(
            use_tc_tiling_on_sc=False, needs_layout_passes=False),
        scratch_types=scratch,
    )(functools.partial(_hop_body, first))


def _attn_agg(o2, w1, w2):
    o = o2.reshape(BATCH, RDIM, DIM)
    u = jnp.sum(w1 * w2[None, :], axis=1)
    att = jnp.maximum(jnp.sum(o * u[None, None, :], axis=-1), 0.0)
    att = att - jnp.max(att, axis=-1, keepdims=True)
    e = jnp.exp(att)
    an = e / jnp.sum(e, axis=-1, keepdims=True)
    return jnp.sum(o * an[..., None], axis=1)


def _tc_hop_body(o_ref, v_ref, w1_ref, w2_ref, t_ref, vn_ref, oa_ref):
    oa = _attn_agg(o_ref[...], w1_ref[...], w2_ref[...])
    oa_ref[...] = oa
    vn_ref[...] = jnp.dot(v_ref[...] + oa, t_ref[...],
                          preferred_element_type=jnp.float32)


def _tc_hop(o2, v, w1, w2, tmat):
    return pl.pallas_call(
        _tc_hop_body,
        out_shape=(jax.ShapeDtypeStruct((BATCH, DIM), jnp.float32),
                   jax.ShapeDtypeStruct((BATCH, DIM), jnp.float32)),
    )(o2, v, w1, w2, tmat)


def _tc_final_body(o_ref, w1_ref, w2_ref, items_ref, neg_ref, oa0_ref,
                   p0_ref, p1_ref, t_ref, out_ref):
    oa1 = _attn_agg(o_ref[...], w1_ref[...], w2_ref[...])
    y = oa0_ref[...] + oa1
    ps = jnp.sum(items_ref[...] * y, axis=1)
    ns = jnp.sum(neg_ref[...] * y, axis=1)
    d = ps - ns
    ls = jnp.minimum(d, 0.0) - jnp.log(1.0 + jnp.exp(-jnp.abs(d)))
    mf = -jnp.sum(ls) / BATCH
    p = p0_ref[...] + p1_ref[...]
    kge = jnp.sum(p[:, 0:16]) / (NGRP * N_MEMORY)
    l2 = jnp.sum(p[:, 16:64]) + jnp.sum(t_ref[...] * t_ref[...])
    out_ref[0] = mf - KGE_W * kge + L2_W * l2


def _tc_final(o2, w1, w2, items, neg, oa0, p0, p1, tmat):
    return pl.pallas_call(
        _tc_final_body,
        out_specs=pl.BlockSpec(memory_space=pltpu.SMEM),
        out_shape=jax.ShapeDtypeStruct((1,), jnp.float32),
    )(o2, w1, w2, items, neg, oa0, p0, p1, tmat)


def kernel(pos_items, neg_items, memories_h, memories_r, memories_t,
           entity_emb, relation_emb, transform_matrix, att_w1, att_w2):
    ent = entity_emb
    rel = relation_emb
    item_idx = jnp.concatenate(
        [pos_items.astype(jnp.int32).reshape(NW, VPW),
         neg_items.astype(jnp.int32).reshape(NW, VPW)], axis=1).reshape(-1)
    w2 = att_w2.reshape(N_HOPS, DIM)

    ih0 = memories_h[0].reshape(-1).astype(jnp.int32)
    it0 = memories_t[0].reshape(-1).astype(jnp.int32)
    mr0 = memories_r[0].reshape(-1).astype(jnp.int32)
    o0, p0, irows = _hop_kernel(True)(ent, ih0, it0, mr0, rel, item_idx)
    ir = irows.reshape(NW, 2, VPW, DIM)
    items = ir[:, 0].reshape(BATCH, DIM)
    negr = ir[:, 1].reshape(BATCH, DIM)

    v1, oa0 = _tc_hop(o0, items, att_w1[0], w2[0], transform_matrix)
    ih1 = memories_h[1].reshape(-1).astype(jnp.int32)
    it1 = memories_t[1].reshape(-1).astype(jnp.int32)
    mr1 = memories_r[1].reshape(-1).astype(jnp.int32)
    o1, p1 = _hop_kernel(False)(ent, ih1, it1, mr1, rel, v1)
    out = _tc_final(o1, att_w1[1], w2[1], items, negr, oa0, p0, p1,
                    transform_matrix)
    return out[0]
